# baseline (speedup 1.0000x reference)
import functools

import jax
import jax.numpy as jnp
from jax import lax
from jax.experimental import pallas as pl
from jax.experimental.pallas import tpu as pltpu
from jax.experimental.pallas import tpu_sc as plsc

NC = 2
NS = 16
NW = NC * NS
CHUNK = 128


def _sc_mesh():
  return plsc.VectorSubcoreMesh(core_axis_name="c", subcore_axis_name="s",
                                num_cores=NC, num_subcores=NS)


def _make_deg_kernel(rpt, npad):
  zrows = npad // NS

  @functools.partial(
      pl.kernel,
      out_type=jax.ShapeDtypeStruct((NC, npad, 16), jnp.float32),
      mesh=_sc_mesh(),
      scratch_types=[
          pltpu.VMEM((rpt, CHUNK), jnp.int32),
          pltpu.VMEM((CHUNK, 16), jnp.float32),
          pltpu.VMEM_SHARED((npad, 16), jnp.float32),
      ],
  )
  def deg_kernel(dst3d, zeros16, ones16, out, dstv, onesv, dacc):
    c = lax.axis_index("c")
    s = lax.axis_index("s")
    w = c * NS + s
    pltpu.sync_copy(dst3d.at[w], dstv)
    pltpu.sync_copy(ones16, onesv)
    pltpu.sync_copy(zeros16, dacc.at[pl.ds(s * zrows, zrows)])
    plsc.subcore_barrier()

    @pl.loop(0, rpt)
    def _(j):
      pltpu.sync_copy(onesv, dacc.at[dstv.at[j]], add=True)

    plsc.subcore_barrier()
    pltpu.sync_copy(dacc.at[pl.ds(s * zrows, zrows)],
                    out.at[c, pl.ds(s * zrows, zrows)])

  return deg_kernel


def _make_agg_kernel(n, rpt, npad):
  zrows = npad // NS

  @functools.partial(
      pl.kernel,
      out_type=jax.ShapeDtypeStruct((NC, npad, 128), jnp.float32),
      mesh=_sc_mesh(),
      scratch_types=[
          pltpu.VMEM((rpt, CHUNK), jnp.int32),
          pltpu.VMEM((rpt, CHUNK), jnp.int32),
          pltpu.VMEM((CHUNK, 128), jnp.float32),
          pltpu.VMEM_SHARED((npad, 128), jnp.float32),
          pltpu.SemaphoreType.DMA,
      ],
  )
  def agg_kernel(table, src3d, dst3d, zeros, out, srcv, dstv, rows, acc, sem):
    c = lax.axis_index("c")
    s = lax.axis_index("s")
    w = c * NS + s
    pltpu.sync_copy(src3d.at[w], srcv)
    pltpu.sync_copy(dst3d.at[w], dstv)
    pltpu.sync_copy(zeros, acc.at[pl.ds(s * zrows, zrows)])
    plsc.subcore_barrier()

    @pl.loop(0, rpt)
    def _(j):
      pltpu.async_copy(table.at[srcv.at[j]], rows, sem).wait()
      pltpu.sync_copy(rows, acc.at[dstv.at[j]], add=True)

    plsc.subcore_barrier()
    pltpu.sync_copy(acc.at[pl.ds(s * zrows, zrows)],
                    out.at[c, pl.ds(s * zrows, zrows)])

  return agg_kernel


def _prep_body(degp_ref, x_ref, xs_ref, dsb_ref):
  d = degp_ref[0, :, 0:1] + degp_ref[1, :, 0:1] + 1.0
  dsb = jnp.broadcast_to(lax.rsqrt(d), xs_ref.shape)
  dsb_ref[...] = dsb
  xs_ref[...] = x_ref[...] * dsb


def _mid_body(p1_ref, xs_ref, dsb_ref, w1t_ref, b1_ref, w2t_ref, t_ref):
  s = (p1_ref[0] + p1_ref[1] + xs_ref[...]) * dsb_ref[...]
  h = jnp.dot(s, w1t_ref[...], preferred_element_type=jnp.float32)
  h = jnp.maximum(h + b1_ref[...], 0.0)
  t = jnp.dot(h, w2t_ref[...], preferred_element_type=jnp.float32)
  t_ref[...] = t * dsb_ref[...]


def _fin_body(p2_ref, t_ref, dsb_ref, b2_ref, wlt_ref, bl_ref, y_ref):
  s = (p2_ref[0] + p2_ref[1] + t_ref[...]) * dsb_ref[...]
  o = jnp.maximum(s + b2_ref[...], 0.0)
  y = jnp.dot(o, wlt_ref[...], preferred_element_type=jnp.float32)
  y_ref[...] = y + bl_ref[...]


def _row_spec(r, d):
  return pl.BlockSpec((r, d), lambda i: (i, 0))


def _part_spec(r, d):
  return pl.BlockSpec((NC, r, d), lambda i: (0, i, 0))


def _full_spec(shape):
  return pl.BlockSpec(shape, lambda i: tuple(0 for _ in shape))


def kernel(x, edge_index, W1, b1, W2, b2, Wl, bl):
  n, d_in = x.shape
  e = edge_index.shape[1]
  d_hid = W1.shape[0]
  d_out = W2.shape[0]

  rpt = -(-e // (NW * CHUNK))
  rpt = -(-rpt // 4) * 4
  e_pad = NW * rpt * CHUNK
  npad = -((n + 1) // -128) * 128
  zrows = npad // NS

  src = edge_index[0]
  dst = edge_index[1]
  pad = e_pad - e
  src3d = jnp.concatenate(
      [src, jnp.zeros((pad,), jnp.int32)]).reshape(NW, rpt, CHUNK)
  dst3d = jnp.concatenate(
      [dst, jnp.full((pad,), n, jnp.int32)]).reshape(NW, rpt, CHUNK)

  zeros16 = jnp.zeros((zrows, 16), jnp.float32)
  ones16 = jnp.ones((CHUNK, 16), jnp.float32)
  zeros128 = jnp.zeros((zrows, 128), jnp.float32)

  deg_k = _make_deg_kernel(rpt, npad)
  agg_k = _make_agg_kernel(n, rpt, npad)

  degp = deg_k(dst3d, zeros16, ones16)

  r = 1000
  grid = (n // r,)

  xs, dsb = pl.pallas_call(
      _prep_body,
      grid=grid,
      in_specs=[_part_spec(r, 16), _row_spec(r, d_in)],
      out_specs=[_row_spec(r, d_in), _row_spec(r, d_in)],
      out_shape=[jax.ShapeDtypeStruct((n, d_in), jnp.float32),
                 jax.ShapeDtypeStruct((n, d_in), jnp.float32)],
  )(degp, x)

  p1 = agg_k(xs, src3d, dst3d, zeros128)

  t = pl.pallas_call(
      _mid_body,
      grid=grid,
      in_specs=[_part_spec(r, d_in), _row_spec(r, d_in), _row_spec(r, d_in),
                _full_spec((d_in, d_hid)), _full_spec((1, d_hid)),
                _full_spec((d_hid, d_out))],
      out_specs=_row_spec(r, d_out),
      out_shape=jax.ShapeDtypeStruct((n, d_out), jnp.float32),
  )(p1, xs, dsb, W1.T, b1.reshape(1, -1), W2.T)

  p2 = agg_k(t, src3d, dst3d, zeros128)

  y = pl.pallas_call(
      _fin_body,
      grid=grid,
      in_specs=[_part_spec(r, d_out), _row_spec(r, d_out), _row_spec(r, d_out),
                _full_spec((1, d_out)), _full_spec((d_out, Wl.shape[0])),
                _full_spec((1, Wl.shape[0]))],
      out_specs=_row_spec(r, Wl.shape[0]),
      out_shape=jax.ShapeDtypeStruct((n, Wl.shape[0]), jnp.float32),
  )(p2, t, dsb, b2.reshape(1, -1), Wl.T, bl.reshape(1, -1))

  return y


# speedup vs baseline: 2.4648x; 2.4648x over previous
import functools

import jax
import jax.numpy as jnp
from jax import lax
from jax.experimental import pallas as pl
from jax.experimental.pallas import tpu as pltpu
from jax.experimental.pallas import tpu_sc as plsc

NC = 2
NS = 16
NW = NC * NS
CHUNK = 128


def _sc_mesh():
  return plsc.VectorSubcoreMesh(core_axis_name="c", subcore_axis_name="s",
                                num_cores=NC, num_subcores=NS)


def _make_deg_kernel(rpt, npad):
  zrows = npad // NS

  @functools.partial(
      pl.kernel,
      out_type=jax.ShapeDtypeStruct((NC, npad, 128), jnp.float32),
      mesh=_sc_mesh(),
      scratch_types=[
          pltpu.VMEM((rpt, CHUNK), jnp.int32),
          pltpu.VMEM((CHUNK, 128), jnp.float32),
          pltpu.VMEM_SHARED((npad, 128), jnp.float32),
      ],
  )
  def deg_kernel(dst3d, zeros, ones, out, dstv, onesv, dacc):
    c = lax.axis_index("c")
    s = lax.axis_index("s")
    w = c * NS + s
    pltpu.sync_copy(dst3d.at[w], dstv)
    pltpu.sync_copy(ones, onesv)
    pltpu.sync_copy(zeros, dacc.at[pl.ds(s * zrows, zrows)])
    plsc.subcore_barrier()

    @pl.loop(0, rpt)
    def _(j):
      pltpu.sync_copy(onesv, dacc.at[dstv.at[j]], add=True)

    plsc.subcore_barrier()
    pltpu.sync_copy(dacc.at[pl.ds(s * zrows, zrows)],
                    out.at[c, pl.ds(s * zrows, zrows)])

  return deg_kernel


def _make_agg_kernel(n, rpt, npad):
  zrows = npad // NS

  @functools.partial(
      pl.kernel,
      out_type=jax.ShapeDtypeStruct((NC, npad, 128), jnp.float32),
      mesh=_sc_mesh(),
      scratch_types=[
          pltpu.VMEM((rpt, CHUNK), jnp.int32),
          pltpu.VMEM((rpt, CHUNK), jnp.int32),
          pltpu.VMEM((CHUNK, 128), jnp.float32),
          pltpu.VMEM_SHARED((npad, 128), jnp.float32),
          pltpu.SemaphoreType.DMA,
      ],
  )
  def agg_kernel(table, src3d, dst3d, zeros, out, srcv, dstv, rows, acc, sem):
    c = lax.axis_index("c")
    s = lax.axis_index("s")
    w = c * NS + s
    pltpu.sync_copy(src3d.at[w], srcv)
    pltpu.sync_copy(dst3d.at[w], dstv)
    pltpu.sync_copy(zeros, acc.at[pl.ds(s * zrows, zrows)])
    plsc.subcore_barrier()

    @pl.loop(0, rpt)
    def _(j):
      pltpu.async_copy(table.at[srcv.at[j]], rows, sem).wait()
      pltpu.sync_copy(rows, acc.at[dstv.at[j]], add=True)

    plsc.subcore_barrier()
    pltpu.sync_copy(acc.at[pl.ds(s * zrows, zrows)],
                    out.at[c, pl.ds(s * zrows, zrows)])

  return agg_kernel


def _prep_body(degp_ref, x_ref, xs_ref, dsb_ref):
  d = degp_ref[0, :, 0:1] + degp_ref[1, :, 0:1] + 1.0
  dsb = jnp.broadcast_to(lax.rsqrt(d), xs_ref.shape)
  dsb_ref[...] = dsb
  xs_ref[...] = x_ref[...] * dsb


def _mid_body(p1_ref, xs_ref, dsb_ref, w1t_ref, b1_ref, w2t_ref, t_ref):
  s = (p1_ref[0] + p1_ref[1] + xs_ref[...]) * dsb_ref[...]
  h = jnp.dot(s, w1t_ref[...], preferred_element_type=jnp.float32)
  h = jnp.maximum(h + b1_ref[...], 0.0)
  t = jnp.dot(h, w2t_ref[...], preferred_element_type=jnp.float32)
  t_ref[...] = t * dsb_ref[...]


def _fin_body(p2_ref, t_ref, dsb_ref, b2_ref, wlt_ref, bl_ref, y_ref):
  s = (p2_ref[0] + p2_ref[1] + t_ref[...]) * dsb_ref[...]
  o = jnp.maximum(s + b2_ref[...], 0.0)
  y = jnp.dot(o, wlt_ref[...], preferred_element_type=jnp.float32)
  y_ref[...] = y + bl_ref[...]


def _row_spec(r, d):
  return pl.BlockSpec((r, d), lambda i: (i, 0))


def _part_spec(r, d):
  return pl.BlockSpec((NC, r, d), lambda i: (0, i, 0))


def _full_spec(shape):
  return pl.BlockSpec(shape, lambda i: tuple(0 for _ in shape))


def kernel(x, edge_index, W1, b1, W2, b2, Wl, bl):
  n, d_in = x.shape
  e = edge_index.shape[1]
  d_hid = W1.shape[0]
  d_out = W2.shape[0]

  rpt = -(-e // (NW * CHUNK))
  e_pad = NW * rpt * CHUNK
  npad = -((n + 1) // -128) * 128
  zrows = npad // NS

  src = edge_index[0]
  dst = edge_index[1]
  pad = e_pad - e
  pad_iota = jnp.arange(pad, dtype=jnp.int32)
  src3d = jnp.concatenate(
      [src, pad_iota % n]).reshape(NW, rpt, CHUNK)
  dst3d = jnp.concatenate(
      [dst, n + pad_iota % (npad - n)]).reshape(NW, rpt, CHUNK)

  zeros128 = jnp.zeros((zrows, 128), jnp.float32)
  ones128 = jnp.ones((CHUNK, 128), jnp.float32)

  deg_k = _make_deg_kernel(rpt, npad)
  agg_k = _make_agg_kernel(n, rpt, npad)

  degp = deg_k(dst3d, zeros128, ones128)

  r = 1000
  grid = (n // r,)

  xs, dsb = pl.pallas_call(
      _prep_body,
      grid=grid,
      in_specs=[_part_spec(r, 128), _row_spec(r, d_in)],
      out_specs=[_row_spec(r, d_in), _row_spec(r, d_in)],
      out_shape=[jax.ShapeDtypeStruct((n, d_in), jnp.float32),
                 jax.ShapeDtypeStruct((n, d_in), jnp.float32)],
  )(degp, x)

  p1 = agg_k(xs, src3d, dst3d, zeros128)

  t = pl.pallas_call(
      _mid_body,
      grid=grid,
      in_specs=[_part_spec(r, d_in), _row_spec(r, d_in), _row_spec(r, d_in),
                _full_spec((d_in, d_hid)), _full_spec((1, d_hid)),
                _full_spec((d_hid, d_out))],
      out_specs=_row_spec(r, d_out),
      out_shape=jax.ShapeDtypeStruct((n, d_out), jnp.float32),
  )(p1, xs, dsb, W1.T, b1.reshape(1, -1), W2.T)

  p2 = agg_k(t, src3d, dst3d, zeros128)

  y = pl.pallas_call(
      _fin_body,
      grid=grid,
      in_specs=[_part_spec(r, d_out), _row_spec(r, d_out), _row_spec(r, d_out),
                _full_spec((1, d_out)), _full_spec((d_out, Wl.shape[0])),
                _full_spec((1, Wl.shape[0]))],
      out_specs=_row_spec(r, Wl.shape[0]),
      out_shape=jax.ShapeDtypeStruct((n, Wl.shape[0]), jnp.float32),
  )(p2, t, dsb, b2.reshape(1, -1), Wl.T, bl.reshape(1, -1))

  return y
